# baseline (speedup 1.0000x reference)
import functools

import jax
import jax.numpy as jnp
from jax import lax
from jax.experimental import pallas as pl
from jax.experimental.pallas import tpu as pltpu
from jax.experimental.pallas import tpu_sc as plsc

B, F, D, T = 4, 2048, 128, 256
K = 15
NW = 32
TPW = (B * T) // NW
LANES = 16
STEP = F // T
SPAN = 272
HALF0 = 152
HALF1 = SPAN - HALF0

_mesh = plsc.VectorSubcoreMesh(core_axis_name="c", subcore_axis_name="s")


@functools.partial(
    pl.kernel,
    mesh=_mesh,
    out_type=jax.ShapeDtypeStruct((B * T * D,), jnp.float32),
    scratch_types=[
        pltpu.VMEM((TPW,), jnp.int32),
        pltpu.VMEM((TPW,), jnp.int32),
        pltpu.VMEM((SPAN * D,), jnp.float32),
        pltpu.VMEM((TPW * D,), jnp.float32),
        pltpu.SemaphoreType.DMA,
        pltpu.SemaphoreType.DMA,
    ],
    compiler_params=pltpu.CompilerParams(
        use_tc_tiling_on_sc=False, needs_layout_passes=False
    ),
)
def _tatum_pool_sc(feat_hbm, starts_hbm, stops_hbm, out_hbm,
                   sv, ev, rowsv, outv, sem0, sem1):
    c = lax.axis_index("c")
    s = lax.axis_index("s")
    w = c * 16 + s
    b = w // (T // TPW)
    t0 = (w % (T // TPW)) * TPW

    base_row = jnp.minimum(b * F + STEP * t0, B * F - SPAN)
    off0 = base_row - b * F

    cp0 = pltpu.async_copy(feat_hbm.at[pl.ds(base_row * D, HALF0 * D)],
                           rowsv.at[pl.ds(0, HALF0 * D)], sem0)
    cp1 = pltpu.async_copy(feat_hbm.at[pl.ds((base_row + HALF0) * D, HALF1 * D)],
                           rowsv.at[pl.ds(HALF0 * D, HALF1 * D)], sem1)

    pltpu.sync_copy(starts_hbm.at[b, pl.ds(t0, TPW)], sv)
    pltpu.sync_copy(stops_hbm.at[b, pl.ds(t0, TPW)], ev)

    def body(d, carry):
        idxs, ovec = carry
        acc = plsc.load_gather(rowsv, [idxs[0]])
        for k in range(1, K):
            acc = jnp.maximum(acc, plsc.load_gather(rowsv, [idxs[k]]))
        plsc.store_scatter(outv, [ovec], acc)
        return tuple(i + 1 for i in idxs), ovec + 1

    cps = (cp0, cp1)
    for g in range(TPW // LANES):
        svec = sv[pl.ds(g * LANES, LANES)]
        emax = ev[pl.ds(g * LANES, LANES)] - 1
        idxs = tuple(
            (jnp.minimum(svec + k, emax) - off0) * D for k in range(K)
        )
        ovec = (lax.iota(jnp.int32, LANES) + g * LANES) * D
        cps[g].wait()
        lax.fori_loop(0, D, body, (idxs, ovec))

    pltpu.sync_copy(outv, out_hbm.at[pl.ds(w * TPW * D, TPW * D)])


def kernel(featureMaps, tatumsBoundaries):
    feat1d = featureMaps.reshape(B * F * D)
    starts = tatumsBoundaries[..., 0].astype(jnp.int32)
    stops = tatumsBoundaries[..., 1].astype(jnp.int32)
    out = _tatum_pool_sc(feat1d, starts, stops)
    return out.reshape(B, T, D)


# speedup vs baseline: 2.1746x; 2.1746x over previous
import functools

import jax
import jax.numpy as jnp
from jax import lax
from jax.experimental import pallas as pl
from jax.experimental.pallas import tpu as pltpu
from jax.experimental.pallas import tpu_sc as plsc

B, F, D, T = 4, 2048, 128, 256
K = 15
NW = 32
TPW = (B * T) // NW
LANES = 16
CPT = D // LANES
STEP = F // T
SPAN = 272
HALF0 = 152
HALF1 = SPAN - HALF0

_mesh = plsc.VectorSubcoreMesh(core_axis_name="c", subcore_axis_name="s")


@functools.partial(
    pl.kernel,
    mesh=_mesh,
    out_type=jax.ShapeDtypeStruct((B * T, D), jnp.float32),
    scratch_types=[
        pltpu.VMEM((TPW,), jnp.int32),
        pltpu.VMEM((TPW,), jnp.int32),
        pltpu.VMEM((SPAN, D), jnp.float32),
        pltpu.VMEM((TPW, D), jnp.float32),
        pltpu.SemaphoreType.DMA,
        pltpu.SemaphoreType.DMA,
    ],
    compiler_params=pltpu.CompilerParams(
        use_tc_tiling_on_sc=False, needs_layout_passes=False
    ),
)
def _tatum_pool_sc(feat_hbm, starts_hbm, stops_hbm, out_hbm,
                   sv, ev, rowsv, outv, sem0, sem1):
    c = lax.axis_index("c")
    s = lax.axis_index("s")
    w = c * 16 + s
    b = w // (T // TPW)
    t0 = (w % (T // TPW)) * TPW

    base_row = jnp.minimum(b * F + STEP * t0, B * F - SPAN)
    off0 = base_row - b * F

    cp0 = pltpu.async_copy(feat_hbm.at[pl.ds(base_row, HALF0)],
                           rowsv.at[pl.ds(0, HALF0)], sem0)
    cp1 = pltpu.async_copy(feat_hbm.at[pl.ds(base_row + HALF0, HALF1)],
                           rowsv.at[pl.ds(HALF0, HALF1)], sem1)

    pltpu.sync_copy(starts_hbm.at[b, pl.ds(t0, TPW)], sv)
    pltpu.sync_copy(stops_hbm.at[b, pl.ds(t0, TPW)], ev)

    def body(t, carry):
        tsplat = jnp.full((LANES,), t, jnp.int32)
        st = jnp.max(plsc.load_gather(sv, [tsplat])) - off0
        em = jnp.max(plsc.load_gather(ev, [tsplat])) - (off0 + 1)
        rk = [jnp.minimum(st + k, em) for k in range(K)]
        for cc in range(CPT):
            acc = rowsv[rk[0], pl.ds(cc * LANES, LANES)]
            for k in range(1, K):
                acc = jnp.maximum(acc, rowsv[rk[k], pl.ds(cc * LANES, LANES)])
            outv[t, pl.ds(cc * LANES, LANES)] = acc
        return carry

    cp0.wait()
    lax.fori_loop(0, TPW // 2, body, 0)
    cp1.wait()
    lax.fori_loop(TPW // 2, TPW, body, 0)

    pltpu.sync_copy(outv, out_hbm.at[pl.ds(w * TPW, TPW)])


def kernel(featureMaps, tatumsBoundaries):
    feat2d = featureMaps.reshape(B * F, D)
    starts = tatumsBoundaries[..., 0].astype(jnp.int32)
    stops = tatumsBoundaries[..., 1].astype(jnp.int32)
    out = _tatum_pool_sc(feat2d, starts, stops)
    return out.reshape(B, T, D)
